# baseline (speedup 1.0000x reference)
import functools

import jax
import jax.numpy as jnp
from jax import lax
from jax.experimental import pallas as pl
from jax.experimental.pallas import tpu as pltpu
from jax.experimental.pallas import tpu_sc as plsc

NN = 10000
NG = 64
HID = 256
NE = 320000
ACC_ROWS = 10240
E_PAD = 327680
EBLK = 128
ROWS_ALL = E_PAD // EBLK
TILES = 16
ZROWS = ACC_ROWS // TILES
ROW_BLK = 2000
GRID = NN // ROW_BLK


def _mesh():
    return plsc.VectorSubcoreMesh(core_axis_name="c", subcore_axis_name="s")


def _deg_body(dst_hbm, ones_hbm, zeros_hbm, out_hbm, dstv, onesv, acc):
    c = lax.axis_index("c")
    s = lax.axis_index("s")
    rpt = ROWS_ALL // 32
    pltpu.sync_copy(zeros_hbm.at[pl.ds(s * ZROWS, ZROWS)],
                    acc.at[pl.ds(s * ZROWS, ZROWS)])
    pltpu.sync_copy(ones_hbm, onesv)
    base = (c * TILES + s) * rpt
    pltpu.sync_copy(dst_hbm.at[pl.ds(base, rpt)], dstv)
    plsc.subcore_barrier()

    def blk(j, carry):
        pltpu.sync_copy(onesv, acc.at[dstv.at[j]], add=True)
        return carry

    lax.fori_loop(0, rpt, blk, 0)
    plsc.subcore_barrier()
    pltpu.sync_copy(acc.at[pl.ds(s * ZROWS, ZROWS)],
                    out_hbm.at[c].at[pl.ds(s * ZROWS, ZROWS)])


@functools.cache
def _deg_call():
    return functools.partial(
        pl.kernel,
        mesh=_mesh(),
        out_type=jax.ShapeDtypeStruct((2, ACC_ROWS, 128), jnp.float32),
        scratch_types=[
            pltpu.VMEM((ROWS_ALL // 32, EBLK), jnp.int32),
            pltpu.VMEM((EBLK, 128), jnp.float32),
            pltpu.VMEM_SHARED((ACC_ROWS, 128), jnp.float32),
        ],
    )(_deg_body)


def _agg_pipeline(ch, tbl_hbm, src_view, dst_view, src_base, dst_base, nrows,
                  srcv, dstv, buf0, buf1, acc, semg0, semg1, sems0, sems1):
    def wg(buf, sem):
        pltpu.make_async_copy(tbl_hbm.at[srcv.at[0]], buf, sem).wait()

    def ws(buf, sem):
        pltpu.make_async_copy(buf, acc.at[dstv.at[0]], sem).wait()

    def chunk(ci, carry):
        pltpu.sync_copy(src_view.at[pl.ds(src_base + ci * ch, ch)], srcv)
        pltpu.sync_copy(dst_view.at[pl.ds(dst_base + ci * ch, ch)], dstv)
        pltpu.async_copy(tbl_hbm.at[srcv.at[0]], buf0, semg0)
        wg(buf0, semg0)
        pltpu.async_copy(buf0, acc.at[dstv.at[0]], sems0, add=True)
        pltpu.async_copy(tbl_hbm.at[srcv.at[1]], buf1, semg1)

        def inner(k, c2):
            j = 2 * k + 1
            wg(buf1, semg1)
            ws(buf0, sems0)
            pltpu.async_copy(buf1, acc.at[dstv.at[j]], sems1, add=True)
            pltpu.async_copy(tbl_hbm.at[srcv.at[j + 1]], buf0, semg0)
            wg(buf0, semg0)
            ws(buf1, sems1)
            pltpu.async_copy(buf0, acc.at[dstv.at[j + 1]], sems0, add=True)
            pltpu.async_copy(tbl_hbm.at[srcv.at[j + 2]], buf1, semg1)
            return c2

        lax.fori_loop(0, ch // 2 - 1, inner, 0)
        wg(buf1, semg1)
        ws(buf0, sems0)
        pltpu.async_copy(buf1, acc.at[dstv.at[ch - 1]], sems1, add=True)
        ws(buf1, sems1)
        return carry

    lax.fori_loop(0, nrows // ch, chunk, 0)


CH_ES = 40
CH_CS = 32


def _scratch(ch):
    return [
        pltpu.VMEM((ch, EBLK), jnp.int32),
        pltpu.VMEM((ch, EBLK), jnp.int32),
        pltpu.VMEM((EBLK, 128), jnp.float32),
        pltpu.VMEM((EBLK, 128), jnp.float32),
        pltpu.VMEM_SHARED((ACC_ROWS, 128), jnp.float32),
        pltpu.SemaphoreType.DMA,
        pltpu.SemaphoreType.DMA,
        pltpu.SemaphoreType.DMA,
        pltpu.SemaphoreType.DMA,
    ]


def _agg_es_body(tbl_hbm, src_hbm, dst_hbm, zeros_hbm, out_hbm,
                 srcv, dstv, buf0, buf1, acc, semg0, semg1, sems0, sems1):
    c = lax.axis_index("c")
    s = lax.axis_index("s")
    rpt = ROWS_ALL // 32
    pltpu.sync_copy(zeros_hbm.at[pl.ds(s * ZROWS, ZROWS)],
                    acc.at[pl.ds(s * ZROWS, ZROWS)])
    plsc.subcore_barrier()
    base = (c * TILES + s) * rpt
    _agg_pipeline(CH_ES, tbl_hbm, src_hbm.at[c], dst_hbm, base, base, rpt,
                  srcv, dstv, buf0, buf1, acc, semg0, semg1, sems0, sems1)
    plsc.subcore_barrier()
    pltpu.sync_copy(acc.at[pl.ds(s * ZROWS, ZROWS)],
                    out_hbm.at[c].at[pl.ds(s * ZROWS, ZROWS)])


@functools.cache
def _agg_es_call():
    return functools.partial(
        pl.kernel,
        mesh=_mesh(),
        out_type=jax.ShapeDtypeStruct((2, ACC_ROWS, 128), jnp.float32),
        scratch_types=_scratch(CH_ES),
    )(_agg_es_body)


def _agg_cs_body(tbl_hbm, src_hbm, dst_hbm, zeros_hbm, out_hbm,
                 srcv, dstv, buf0, buf1, acc, semg0, semg1, sems0, sems1):
    c = lax.axis_index("c")
    s = lax.axis_index("s")
    rpt = ROWS_ALL // TILES
    pltpu.sync_copy(zeros_hbm.at[pl.ds(s * ZROWS, ZROWS)],
                    acc.at[pl.ds(s * ZROWS, ZROWS)])
    plsc.subcore_barrier()
    _agg_pipeline(CH_CS, tbl_hbm, src_hbm.at[c], dst_hbm, s * rpt, s * rpt,
                  rpt, srcv, dstv, buf0, buf1, acc, semg0, semg1, sems0, sems1)
    plsc.subcore_barrier()
    pltpu.sync_copy(acc.at[pl.ds(s * ZROWS, ZROWS)],
                    out_hbm.at[c].at[pl.ds(s * ZROWS, ZROWS)])


@functools.cache
def _agg_cs_call():
    return functools.partial(
        pl.kernel,
        mesh=_mesh(),
        out_type=jax.ShapeDtypeStruct((2, ACC_ROWS, 128), jnp.float32),
        scratch_types=_scratch(CH_CS),
    )(_agg_cs_body)


def _prep_body(deg_ref, x_ref, dinv_ref, z_ref):
    deg = deg_ref[0, :, 0:1] + deg_ref[1, :, 0:1] + 1.0
    dinv = lax.rsqrt(deg)
    dinv_ref[...] = jnp.broadcast_to(dinv, (ROW_BLK, 128))
    z = dinv * x_ref[...]
    z_ref[0] = z
    z_ref[1] = z


def _prep_call(degp, x):
    return pl.pallas_call(
        _prep_body,
        grid=(GRID,),
        in_specs=[
            pl.BlockSpec((2, ROW_BLK, 128), lambda i: (0, i, 0)),
            pl.BlockSpec((ROW_BLK, 128), lambda i: (i, 0)),
        ],
        out_specs=[
            pl.BlockSpec((ROW_BLK, 128), lambda i: (i, 0)),
            pl.BlockSpec((2, ROW_BLK, 128), lambda i: (0, i, 0)),
        ],
        out_shape=[
            jax.ShapeDtypeStruct((NN, 128), jnp.float32),
            jax.ShapeDtypeStruct((2, NN, 128), jnp.float32),
        ],
    )(degp, x)


def _layer1_body(acc_ref, z_ref, dinv_ref, w_ref, b_ref, out_ref):
    dinv = dinv_ref[:, 0:1]
    u = dinv * (acc_ref[0] + acc_ref[1] + z_ref[0])
    pre = jnp.dot(u, w_ref[...], preferred_element_type=jnp.float32) + b_ref[0]
    gn = dinv * jnp.maximum(pre, 0.0)
    out_ref[0] = gn[:, :128]
    out_ref[1] = gn[:, 128:]


def _layer1_call(acc, z2, dinv128, w, b8):
    return pl.pallas_call(
        _layer1_body,
        grid=(GRID,),
        in_specs=[
            pl.BlockSpec((2, ROW_BLK, 128), lambda i: (0, i, 0)),
            pl.BlockSpec((2, ROW_BLK, 128), lambda i: (0, i, 0)),
            pl.BlockSpec((ROW_BLK, 128), lambda i: (i, 0)),
            pl.BlockSpec((128, HID), lambda i: (0, 0)),
            pl.BlockSpec((8, HID), lambda i: (0, 0)),
        ],
        out_specs=pl.BlockSpec((2, ROW_BLK, 128), lambda i: (0, i, 0)),
        out_shape=jax.ShapeDtypeStruct((2, NN, 128), jnp.float32),
    )(acc, z2, dinv128, w, b8)


def _layer2_body(acc_ref, g_ref, dinv_ref, w_ref, b_ref, out_ref):
    dinv = dinv_ref[:, 0:1]
    u0 = dinv * (acc_ref[0] + g_ref[0])
    u1 = dinv * (acc_ref[1] + g_ref[1])
    pre = (jnp.dot(u0, w_ref[0], preferred_element_type=jnp.float32)
           + jnp.dot(u1, w_ref[1], preferred_element_type=jnp.float32)
           + b_ref[0])
    gn = dinv * jnp.maximum(pre, 0.0)
    out_ref[0] = gn[:, :128]
    out_ref[1] = gn[:, 128:]


def _layer2_call(acc, g, dinv128, w2, b8):
    return pl.pallas_call(
        _layer2_body,
        grid=(GRID,),
        in_specs=[
            pl.BlockSpec((2, ROW_BLK, 128), lambda i: (0, i, 0)),
            pl.BlockSpec((2, ROW_BLK, 128), lambda i: (0, i, 0)),
            pl.BlockSpec((ROW_BLK, 128), lambda i: (i, 0)),
            pl.BlockSpec((2, 128, HID), lambda i: (0, 0, 0)),
            pl.BlockSpec((8, HID), lambda i: (0, 0)),
        ],
        out_specs=pl.BlockSpec((2, ROW_BLK, 128), lambda i: (0, i, 0)),
        out_shape=jax.ShapeDtypeStruct((2, NN, 128), jnp.float32),
    )(acc, g, dinv128, w2, b8)


def _final_body(acc_ref, g_ref, dinv_ref, w_ref, b_ref, batch_ref,
                wf1_ref, bf1_ref, wf2_ref, bf2_ref,
                sums_ref, counts_ref, out_ref):
    i = pl.program_id(0)
    dinv = dinv_ref[:, 0:1]
    u0 = dinv * (acc_ref[0] + g_ref[0])
    u1 = dinv * (acc_ref[1] + g_ref[1])
    pre = (jnp.dot(u0, w_ref[0], preferred_element_type=jnp.float32)
           + jnp.dot(u1, w_ref[1], preferred_element_type=jnp.float32)
           + b_ref[0])
    h = jnp.maximum(pre, 0.0)
    seg = batch_ref[:, 0:1]
    gid = lax.broadcasted_iota(jnp.int32, (ROW_BLK, NG), 1)
    mask = (seg == gid).astype(jnp.float32)
    psum = lax.dot_general(mask, h, (((0,), (0,)), ((), ())),
                           preferred_element_type=jnp.float32)
    pcnt = lax.dot_general(mask, jnp.ones((ROW_BLK, HID), jnp.float32),
                           (((0,), (0,)), ((), ())),
                           preferred_element_type=jnp.float32)

    @pl.when(i == 0)
    def _():
        sums_ref[...] = jnp.zeros_like(sums_ref)
        counts_ref[...] = jnp.zeros_like(counts_ref)
        out_ref[...] = jnp.zeros_like(out_ref)

    sums_ref[...] += psum
    counts_ref[...] += pcnt

    @pl.when(i == GRID - 1)
    def _():
        pooled = sums_ref[...] / jnp.maximum(counts_ref[...], 1.0)
        t = jnp.maximum(
            jnp.dot(pooled, wf1_ref[...], preferred_element_type=jnp.float32)
            + bf1_ref[0], 0.0)
        out_ref[...] = (jnp.dot(t, wf2_ref[...],
                                preferred_element_type=jnp.float32)
                        + bf2_ref[0, 0])


def _final_call(acc, g, dinv128, w2, b8, batchr, wf1p, bf1p, wf2p, bf2p):
    return pl.pallas_call(
        _final_body,
        grid=(GRID,),
        in_specs=[
            pl.BlockSpec((2, ROW_BLK, 128), lambda i: (0, i, 0)),
            pl.BlockSpec((2, ROW_BLK, 128), lambda i: (0, i, 0)),
            pl.BlockSpec((ROW_BLK, 128), lambda i: (i, 0)),
            pl.BlockSpec((2, 128, HID), lambda i: (0, 0, 0)),
            pl.BlockSpec((8, HID), lambda i: (0, 0)),
            pl.BlockSpec((ROW_BLK, 128), lambda i: (i, 0)),
            pl.BlockSpec((HID, 128), lambda i: (0, 0)),
            pl.BlockSpec((8, 128), lambda i: (0, 0)),
            pl.BlockSpec((128, 128), lambda i: (0, 0)),
            pl.BlockSpec((8, 128), lambda i: (0, 0)),
        ],
        out_specs=[
            pl.BlockSpec((NG, HID), lambda i: (0, 0)),
            pl.BlockSpec((NG, HID), lambda i: (0, 0)),
            pl.BlockSpec((NG, 128), lambda i: (0, 0)),
        ],
        out_shape=[
            jax.ShapeDtypeStruct((NG, HID), jnp.float32),
            jax.ShapeDtypeStruct((NG, HID), jnp.float32),
            jax.ShapeDtypeStruct((NG, 128), jnp.float32),
        ],
    )(acc, g, dinv128, w2, b8, batchr, wf1p, bf1p, wf2p, bf2p)


def kernel(x, edge_index, batch, W1, b1, W2, b2, W3, b3, Wf1, bf1, Wf2, bf2):
    f32 = jnp.float32
    src = edge_index[0].astype(jnp.int32)
    dst = edge_index[1].astype(jnp.int32)
    pad = E_PAD - NE
    src_p = jnp.concatenate(
        [src, jnp.zeros((pad,), jnp.int32)]).reshape(ROWS_ALL, EBLK)
    dst_p = jnp.concatenate(
        [dst, jnp.full((pad,), NN, jnp.int32)]).reshape(ROWS_ALL, EBLK)
    src2 = jnp.stack([src_p, src_p + NN])

    zeros128 = jnp.zeros((ACC_ROWS, 128), f32)
    ones128 = jnp.ones((EBLK, 128), f32)

    def b8(b):
        return jnp.broadcast_to(b[None, :], (8, b.shape[0]))

    batchr = jnp.broadcast_to(batch.astype(jnp.int32)[:, None], (NN, 128))
    wf1p = jnp.pad(Wf1, ((0, 0), (0, 128 - Wf1.shape[1])))
    bf1p = b8(jnp.pad(bf1, (0, 128 - bf1.shape[0])))
    wf2p = jnp.pad(Wf2, ((0, 128 - Wf2.shape[0]), (0, 128 - Wf2.shape[1])))
    bf2p = b8(jnp.broadcast_to(bf2, (128,)))

    degp = _deg_call()(dst_p, ones128, zeros128)
    dinv128, z2 = _prep_call(degp, x)

    acc1 = _agg_es_call()(z2.reshape(2 * NN, 128), src2, dst_p, zeros128)
    g2 = _layer1_call(acc1, z2, dinv128, W1, b8(b1))

    acc2 = _agg_cs_call()(g2.reshape(2 * NN, 128), src2, dst_p, zeros128)
    g3 = _layer2_call(acc2, g2, dinv128, W2.reshape(2, 128, HID), b8(b2))

    acc3 = _agg_cs_call()(g3.reshape(2 * NN, 128), src2, dst_p, zeros128)
    _, _, out = _final_call(acc3, g3, dinv128, W3.reshape(2, 128, HID),
                            b8(b3), batchr, wf1p, bf1p, wf2p, bf2p)
    return out[:, 0]


# speedup vs baseline: 1.0406x; 1.0406x over previous
import functools

import jax
import jax.numpy as jnp
from jax import lax
from jax.experimental import pallas as pl
from jax.experimental.pallas import tpu as pltpu
from jax.experimental.pallas import tpu_sc as plsc

NN = 10000
NG = 64
HID = 256
NE = 320000
ACC_ROWS = 10240
E_PAD = 327680
EBLK = 128
ROWS_ALL = E_PAD // EBLK
TILES = 16
ZROWS = ACC_ROWS // TILES
ROW_BLK = 2000
GRID = NN // ROW_BLK


def _mesh():
    return plsc.VectorSubcoreMesh(core_axis_name="c", subcore_axis_name="s")


def _deg_body(dst_hbm, ones_hbm, zeros_hbm, out_hbm, dstv, onesv, acc):
    c = lax.axis_index("c")
    s = lax.axis_index("s")
    rpt = ROWS_ALL // 32
    pltpu.sync_copy(zeros_hbm.at[pl.ds(s * ZROWS, ZROWS)],
                    acc.at[pl.ds(s * ZROWS, ZROWS)])
    pltpu.sync_copy(ones_hbm, onesv)
    base = (c * TILES + s) * rpt
    pltpu.sync_copy(dst_hbm.at[pl.ds(base, rpt)], dstv)
    plsc.subcore_barrier()

    def blk(j, carry):
        pltpu.sync_copy(onesv, acc.at[dstv.at[j]], add=True)
        return carry

    lax.fori_loop(0, rpt, blk, 0)
    plsc.subcore_barrier()
    pltpu.sync_copy(acc.at[pl.ds(s * ZROWS, ZROWS)],
                    out_hbm.at[c].at[pl.ds(s * ZROWS, ZROWS)])


@functools.cache
def _deg_call():
    return functools.partial(
        pl.kernel,
        mesh=_mesh(),
        out_type=jax.ShapeDtypeStruct((2, ACC_ROWS, 128), jnp.float32),
        scratch_types=[
            pltpu.VMEM((ROWS_ALL // 32, EBLK), jnp.int32),
            pltpu.VMEM((EBLK, 128), jnp.float32),
            pltpu.VMEM_SHARED((ACC_ROWS, 128), jnp.float32),
        ],
    )(_deg_body)


def _agg_pipeline(ch, tbl_hbm, src_view, dst_view, src_base, dst_base, nrows,
                  srcv, dstv, buf0, buf1, acc, semg0, semg1, sems0, sems1):
    def wg(buf, sem):
        pltpu.make_async_copy(tbl_hbm.at[srcv.at[0]], buf, sem).wait()

    def ws(buf, sem):
        pltpu.make_async_copy(buf, acc.at[dstv.at[0]], sem).wait()

    def chunk(ci, carry):
        pltpu.sync_copy(src_view.at[pl.ds(src_base + ci * ch, ch)], srcv)
        pltpu.sync_copy(dst_view.at[pl.ds(dst_base + ci * ch, ch)], dstv)
        pltpu.async_copy(tbl_hbm.at[srcv.at[0]], buf0, semg0)
        wg(buf0, semg0)
        pltpu.async_copy(buf0, acc.at[dstv.at[0]], sems0, add=True)
        pltpu.async_copy(tbl_hbm.at[srcv.at[1]], buf1, semg1)

        def inner(k, c2):
            j = 2 * k + 1
            wg(buf1, semg1)
            ws(buf0, sems0)
            pltpu.async_copy(buf1, acc.at[dstv.at[j]], sems1, add=True)
            pltpu.async_copy(tbl_hbm.at[srcv.at[j + 1]], buf0, semg0)
            wg(buf0, semg0)
            ws(buf1, sems1)
            pltpu.async_copy(buf0, acc.at[dstv.at[j + 1]], sems0, add=True)
            pltpu.async_copy(tbl_hbm.at[srcv.at[j + 2]], buf1, semg1)
            return c2

        lax.fori_loop(0, ch // 2 - 1, inner, 0)
        wg(buf1, semg1)
        ws(buf0, sems0)
        pltpu.async_copy(buf1, acc.at[dstv.at[ch - 1]], sems1, add=True)
        ws(buf1, sems1)
        return carry

    lax.fori_loop(0, nrows // ch, chunk, 0)


CH_ES = 40
CH_CS = 32


def _scratch(ch):
    return [
        pltpu.VMEM((ch, EBLK), jnp.int32),
        pltpu.VMEM((ch, EBLK), jnp.int32),
        pltpu.VMEM((EBLK, 128), jnp.float32),
        pltpu.VMEM((EBLK, 128), jnp.float32),
        pltpu.VMEM_SHARED((ACC_ROWS, 128), jnp.float32),
        pltpu.SemaphoreType.DMA,
        pltpu.SemaphoreType.DMA,
        pltpu.SemaphoreType.DMA,
        pltpu.SemaphoreType.DMA,
    ]


def _agg_es_body(tbl_hbm, src_hbm, dst_hbm, zeros_hbm, out_hbm,
                 srcv, dstv, buf0, buf1, acc, semg0, semg1, sems0, sems1):
    c = lax.axis_index("c")
    s = lax.axis_index("s")
    rpt = ROWS_ALL // 32
    pltpu.sync_copy(zeros_hbm.at[pl.ds(s * ZROWS, ZROWS)],
                    acc.at[pl.ds(s * ZROWS, ZROWS)])
    plsc.subcore_barrier()
    base = (c * TILES + s) * rpt
    _agg_pipeline(CH_ES, tbl_hbm, src_hbm, dst_hbm, base, base, rpt,
                  srcv, dstv, buf0, buf1, acc, semg0, semg1, sems0, sems1)
    plsc.subcore_barrier()
    pltpu.sync_copy(acc.at[pl.ds(s * ZROWS, ZROWS)],
                    out_hbm.at[c].at[pl.ds(s * ZROWS, ZROWS)])


@functools.cache
def _agg_es_call():
    return functools.partial(
        pl.kernel,
        mesh=_mesh(),
        out_type=jax.ShapeDtypeStruct((2, ACC_ROWS, 128), jnp.float32),
        scratch_types=_scratch(CH_ES),
    )(_agg_es_body)


def _agg_cs_body(tbl_hbm, src_hbm, dst_hbm, zeros_hbm, out_hbm,
                 srcv, dstv, buf0, buf1, acc, semg0, semg1, sems0, sems1):
    c = lax.axis_index("c")
    s = lax.axis_index("s")
    rpt = ROWS_ALL // TILES
    pltpu.sync_copy(zeros_hbm.at[pl.ds(s * ZROWS, ZROWS)],
                    acc.at[pl.ds(s * ZROWS, ZROWS)])
    plsc.subcore_barrier()
    _agg_pipeline(CH_CS, tbl_hbm, src_hbm.at[c], dst_hbm, s * rpt, s * rpt,
                  rpt, srcv, dstv, buf0, buf1, acc, semg0, semg1, sems0, sems1)
    plsc.subcore_barrier()
    pltpu.sync_copy(acc.at[pl.ds(s * ZROWS, ZROWS)],
                    out_hbm.at[c].at[pl.ds(s * ZROWS, ZROWS)])


@functools.cache
def _agg_cs_call():
    return functools.partial(
        pl.kernel,
        mesh=_mesh(),
        out_type=jax.ShapeDtypeStruct((2, ACC_ROWS, 128), jnp.float32),
        scratch_types=_scratch(CH_CS),
    )(_agg_cs_body)


def _prep_body(deg_ref, x_ref, dinv_ref, z_ref):
    deg = deg_ref[0, :, 0:1] + deg_ref[1, :, 0:1] + 1.0
    dinv = lax.rsqrt(deg)
    dinv_ref[...] = jnp.broadcast_to(dinv, (ROW_BLK, 128))
    z_ref[...] = dinv * x_ref[...]


def _prep_call(degp, x):
    return pl.pallas_call(
        _prep_body,
        grid=(GRID,),
        in_specs=[
            pl.BlockSpec((2, ROW_BLK, 128), lambda i: (0, i, 0)),
            pl.BlockSpec((ROW_BLK, 128), lambda i: (i, 0)),
        ],
        out_specs=[
            pl.BlockSpec((ROW_BLK, 128), lambda i: (i, 0)),
            pl.BlockSpec((ROW_BLK, 128), lambda i: (i, 0)),
        ],
        out_shape=[
            jax.ShapeDtypeStruct((NN, 128), jnp.float32),
            jax.ShapeDtypeStruct((NN, 128), jnp.float32),
        ],
    )(degp, x)


def _layer1_body(acc_ref, z_ref, dinv_ref, w_ref, b_ref, out_ref):
    dinv = dinv_ref[:, 0:1]
    u = dinv * (acc_ref[0] + acc_ref[1] + z_ref[...])
    pre = jnp.dot(u, w_ref[...], preferred_element_type=jnp.float32) + b_ref[0]
    gn = dinv * jnp.maximum(pre, 0.0)
    out_ref[0] = gn[:, :128]
    out_ref[1] = gn[:, 128:]


def _layer1_call(acc, z, dinv128, w, b8):
    return pl.pallas_call(
        _layer1_body,
        grid=(GRID,),
        in_specs=[
            pl.BlockSpec((2, ROW_BLK, 128), lambda i: (0, i, 0)),
            pl.BlockSpec((ROW_BLK, 128), lambda i: (i, 0)),
            pl.BlockSpec((ROW_BLK, 128), lambda i: (i, 0)),
            pl.BlockSpec((128, HID), lambda i: (0, 0)),
            pl.BlockSpec((8, HID), lambda i: (0, 0)),
        ],
        out_specs=pl.BlockSpec((2, ROW_BLK, 128), lambda i: (0, i, 0)),
        out_shape=jax.ShapeDtypeStruct((2, NN, 128), jnp.float32),
    )(acc, z, dinv128, w, b8)


def _layer2_body(acc_ref, g_ref, dinv_ref, w_ref, b_ref, out_ref):
    dinv = dinv_ref[:, 0:1]
    u0 = dinv * (acc_ref[0] + g_ref[0])
    u1 = dinv * (acc_ref[1] + g_ref[1])
    pre = (jnp.dot(u0, w_ref[0], preferred_element_type=jnp.float32)
           + jnp.dot(u1, w_ref[1], preferred_element_type=jnp.float32)
           + b_ref[0])
    gn = dinv * jnp.maximum(pre, 0.0)
    out_ref[0] = gn[:, :128]
    out_ref[1] = gn[:, 128:]


def _layer2_call(acc, g, dinv128, w2, b8):
    return pl.pallas_call(
        _layer2_body,
        grid=(GRID,),
        in_specs=[
            pl.BlockSpec((2, ROW_BLK, 128), lambda i: (0, i, 0)),
            pl.BlockSpec((2, ROW_BLK, 128), lambda i: (0, i, 0)),
            pl.BlockSpec((ROW_BLK, 128), lambda i: (i, 0)),
            pl.BlockSpec((2, 128, HID), lambda i: (0, 0, 0)),
            pl.BlockSpec((8, HID), lambda i: (0, 0)),
        ],
        out_specs=pl.BlockSpec((2, ROW_BLK, 128), lambda i: (0, i, 0)),
        out_shape=jax.ShapeDtypeStruct((2, NN, 128), jnp.float32),
    )(acc, g, dinv128, w2, b8)


def _final_body(acc_ref, g_ref, dinv_ref, w_ref, b_ref, batch_ref,
                wf1_ref, bf1_ref, wf2_ref, bf2_ref,
                sums_ref, counts_ref, out_ref):
    i = pl.program_id(0)
    dinv = dinv_ref[:, 0:1]
    u0 = dinv * (acc_ref[0] + g_ref[0])
    u1 = dinv * (acc_ref[1] + g_ref[1])
    pre = (jnp.dot(u0, w_ref[0], preferred_element_type=jnp.float32)
           + jnp.dot(u1, w_ref[1], preferred_element_type=jnp.float32)
           + b_ref[0])
    h = jnp.maximum(pre, 0.0)
    seg = batch_ref[:, 0:1]
    gid = lax.broadcasted_iota(jnp.int32, (ROW_BLK, NG), 1)
    mask = (seg == gid).astype(jnp.float32)
    psum = lax.dot_general(mask, h, (((0,), (0,)), ((), ())),
                           preferred_element_type=jnp.float32)
    pcnt = lax.dot_general(mask, jnp.ones((ROW_BLK, HID), jnp.float32),
                           (((0,), (0,)), ((), ())),
                           preferred_element_type=jnp.float32)

    @pl.when(i == 0)
    def _():
        sums_ref[...] = jnp.zeros_like(sums_ref)
        counts_ref[...] = jnp.zeros_like(counts_ref)
        out_ref[...] = jnp.zeros_like(out_ref)

    sums_ref[...] += psum
    counts_ref[...] += pcnt

    @pl.when(i == GRID - 1)
    def _():
        pooled = sums_ref[...] / jnp.maximum(counts_ref[...], 1.0)
        t = jnp.maximum(
            jnp.dot(pooled, wf1_ref[...], preferred_element_type=jnp.float32)
            + bf1_ref[0], 0.0)
        out_ref[...] = (jnp.dot(t, wf2_ref[...],
                                preferred_element_type=jnp.float32)
                        + bf2_ref[0, 0])


def _final_call(acc, g, dinv128, w2, b8, batchr, wf1p, bf1p, wf2p, bf2p):
    return pl.pallas_call(
        _final_body,
        grid=(GRID,),
        in_specs=[
            pl.BlockSpec((2, ROW_BLK, 128), lambda i: (0, i, 0)),
            pl.BlockSpec((2, ROW_BLK, 128), lambda i: (0, i, 0)),
            pl.BlockSpec((ROW_BLK, 128), lambda i: (i, 0)),
            pl.BlockSpec((2, 128, HID), lambda i: (0, 0, 0)),
            pl.BlockSpec((8, HID), lambda i: (0, 0)),
            pl.BlockSpec((ROW_BLK, 128), lambda i: (i, 0)),
            pl.BlockSpec((HID, 128), lambda i: (0, 0)),
            pl.BlockSpec((8, 128), lambda i: (0, 0)),
            pl.BlockSpec((128, 128), lambda i: (0, 0)),
            pl.BlockSpec((8, 128), lambda i: (0, 0)),
        ],
        out_specs=[
            pl.BlockSpec((NG, HID), lambda i: (0, 0)),
            pl.BlockSpec((NG, HID), lambda i: (0, 0)),
            pl.BlockSpec((NG, 128), lambda i: (0, 0)),
        ],
        out_shape=[
            jax.ShapeDtypeStruct((NG, HID), jnp.float32),
            jax.ShapeDtypeStruct((NG, HID), jnp.float32),
            jax.ShapeDtypeStruct((NG, 128), jnp.float32),
        ],
    )(acc, g, dinv128, w2, b8, batchr, wf1p, bf1p, wf2p, bf2p)


def kernel(x, edge_index, batch, W1, b1, W2, b2, W3, b3, Wf1, bf1, Wf2, bf2):
    f32 = jnp.float32
    src = edge_index[0].astype(jnp.int32)
    dst = edge_index[1].astype(jnp.int32)
    pad = E_PAD - NE
    src_p = jnp.concatenate(
        [src, jnp.zeros((pad,), jnp.int32)]).reshape(ROWS_ALL, EBLK)
    dst_p = jnp.concatenate(
        [dst, jnp.full((pad,), NN, jnp.int32)]).reshape(ROWS_ALL, EBLK)
    src2 = jnp.stack([src_p, src_p + NN])

    zeros128 = jnp.zeros((ACC_ROWS, 128), f32)
    ones128 = jnp.ones((EBLK, 128), f32)

    def b8(b):
        return jnp.broadcast_to(b[None, :], (8, b.shape[0]))

    batchr = jnp.broadcast_to(batch.astype(jnp.int32)[:, None], (NN, 128))
    wf1p = jnp.pad(Wf1, ((0, 0), (0, 128 - Wf1.shape[1])))
    bf1p = b8(jnp.pad(bf1, (0, 128 - bf1.shape[0])))
    wf2p = jnp.pad(Wf2, ((0, 128 - Wf2.shape[0]), (0, 128 - Wf2.shape[1])))
    bf2p = b8(jnp.broadcast_to(bf2, (128,)))

    degp = _deg_call()(dst_p, ones128, zeros128)
    dinv128, z = _prep_call(degp, x)

    acc1 = _agg_es_call()(z, src_p, dst_p, zeros128)
    g2 = _layer1_call(acc1, z, dinv128, W1, b8(b1))

    acc2 = _agg_cs_call()(g2.reshape(2 * NN, 128), src2, dst_p, zeros128)
    g3 = _layer2_call(acc2, g2, dinv128, W2.reshape(2, 128, HID), b8(b2))

    acc3 = _agg_cs_call()(g3.reshape(2 * NN, 128), src2, dst_p, zeros128)
    _, _, out = _final_call(acc3, g3, dinv128, W3.reshape(2, 128, HID),
                            b8(b3), batchr, wf1p, bf1p, wf2p, bf2p)
    return out[:, 0]


# speedup vs baseline: 1.0425x; 1.0019x over previous
import functools

import jax
import jax.numpy as jnp
from jax import lax
from jax.experimental import pallas as pl
from jax.experimental.pallas import tpu as pltpu
from jax.experimental.pallas import tpu_sc as plsc

NN = 10000
NG = 64
HID = 256
NE = 320000
ACC_ROWS = 10240
E_PAD = 327680
EBLK = 128
ROWS_ALL = E_PAD // EBLK
TILES = 16
ZROWS = ACC_ROWS // TILES
ROW_BLK = 2000
GRID = NN // ROW_BLK


def _mesh():
    return plsc.VectorSubcoreMesh(core_axis_name="c", subcore_axis_name="s")


def _deg_body(dst_hbm, ones_hbm, zeros_hbm, out_hbm, dstv, onesv, acc):
    c = lax.axis_index("c")
    s = lax.axis_index("s")
    rpt = ROWS_ALL // 32
    pltpu.sync_copy(zeros_hbm.at[pl.ds(s * ZROWS, ZROWS)],
                    acc.at[pl.ds(s * ZROWS, ZROWS)])
    pltpu.sync_copy(ones_hbm, onesv)
    base = (c * TILES + s) * rpt
    pltpu.sync_copy(dst_hbm.at[pl.ds(base, rpt)], dstv)
    plsc.subcore_barrier()

    def blk(j, carry):
        pltpu.sync_copy(onesv, acc.at[dstv.at[j]], add=True)
        return carry

    lax.fori_loop(0, rpt, blk, 0)
    plsc.subcore_barrier()
    pltpu.sync_copy(acc.at[pl.ds(s * ZROWS, ZROWS)],
                    out_hbm.at[c].at[pl.ds(s * ZROWS, ZROWS)])


@functools.cache
def _deg_call():
    return functools.partial(
        pl.kernel,
        mesh=_mesh(),
        out_type=jax.ShapeDtypeStruct((2, ACC_ROWS, 128), jnp.float32),
        scratch_types=[
            pltpu.VMEM((ROWS_ALL // 32, EBLK), jnp.int32),
            pltpu.VMEM((EBLK, 128), jnp.float32),
            pltpu.VMEM_SHARED((ACC_ROWS, 128), jnp.float32),
        ],
    )(_deg_body)


def _agg_pipeline(ch, tbl_hbm, src_view, dst_view, src_base, dst_base, nrows,
                  srcv, dstv, buf0, buf1, acc, semg0, semg1, sems0, sems1):
    def wg(buf, sem):
        pltpu.make_async_copy(tbl_hbm.at[srcv.at[0]], buf, sem).wait()

    def ws(buf, sem):
        pltpu.make_async_copy(buf, acc.at[dstv.at[0]], sem).wait()

    def chunk(ci, carry):
        pltpu.sync_copy(src_view.at[pl.ds(src_base + ci * ch, ch)], srcv)
        pltpu.sync_copy(dst_view.at[pl.ds(dst_base + ci * ch, ch)], dstv)
        pltpu.async_copy(tbl_hbm.at[srcv.at[0]], buf0, semg0)
        wg(buf0, semg0)
        pltpu.async_copy(buf0, acc.at[dstv.at[0]], sems0, add=True)
        pltpu.async_copy(tbl_hbm.at[srcv.at[1]], buf1, semg1)

        def inner(k, c2):
            j = 2 * k + 1
            wg(buf1, semg1)
            ws(buf0, sems0)
            pltpu.async_copy(buf1, acc.at[dstv.at[j]], sems1, add=True)
            pltpu.async_copy(tbl_hbm.at[srcv.at[j + 1]], buf0, semg0)
            wg(buf0, semg0)
            ws(buf1, sems1)
            pltpu.async_copy(buf0, acc.at[dstv.at[j + 1]], sems0, add=True)
            pltpu.async_copy(tbl_hbm.at[srcv.at[j + 2]], buf1, semg1)
            return c2

        lax.fori_loop(0, ch // 2 - 1, inner, 0)
        wg(buf1, semg1)
        ws(buf0, sems0)
        pltpu.async_copy(buf1, acc.at[dstv.at[ch - 1]], sems1, add=True)
        ws(buf1, sems1)
        return carry

    lax.fori_loop(0, nrows // ch, chunk, 0)


CH_ES = 40
CH_CS = 40


def _scratch(ch):
    return [
        pltpu.VMEM((ch, EBLK), jnp.int32),
        pltpu.VMEM((ch, EBLK), jnp.int32),
        pltpu.VMEM((EBLK, 128), jnp.float32),
        pltpu.VMEM((EBLK, 128), jnp.float32),
        pltpu.VMEM_SHARED((ACC_ROWS, 128), jnp.float32),
        pltpu.SemaphoreType.DMA,
        pltpu.SemaphoreType.DMA,
        pltpu.SemaphoreType.DMA,
        pltpu.SemaphoreType.DMA,
    ]


def _agg_es_body(tbl_hbm, src_hbm, dst_hbm, zeros_hbm, out_hbm,
                 srcv, dstv, buf0, buf1, acc, semg0, semg1, sems0, sems1):
    c = lax.axis_index("c")
    s = lax.axis_index("s")
    rpt = ROWS_ALL // 32
    pltpu.sync_copy(zeros_hbm.at[pl.ds(s * ZROWS, ZROWS)],
                    acc.at[pl.ds(s * ZROWS, ZROWS)])
    plsc.subcore_barrier()
    base = (c * TILES + s) * rpt
    _agg_pipeline(CH_ES, tbl_hbm, src_hbm, dst_hbm, base, base, rpt,
                  srcv, dstv, buf0, buf1, acc, semg0, semg1, sems0, sems1)
    plsc.subcore_barrier()
    pltpu.sync_copy(acc.at[pl.ds(s * ZROWS, ZROWS)],
                    out_hbm.at[c].at[pl.ds(s * ZROWS, ZROWS)])


@functools.cache
def _agg_es_call():
    return functools.partial(
        pl.kernel,
        mesh=_mesh(),
        out_type=jax.ShapeDtypeStruct((2, ACC_ROWS, 128), jnp.float32),
        scratch_types=_scratch(CH_ES),
    )(_agg_es_body)


def _agg_cs_body(tbl_hbm, src_hbm, dst_hbm, zeros_hbm, out_hbm,
                 srcv, dstv, buf0, buf1, acc, semg0, semg1, sems0, sems1):
    c = lax.axis_index("c")
    s = lax.axis_index("s")
    rpt = ROWS_ALL // TILES
    pltpu.sync_copy(zeros_hbm.at[pl.ds(s * ZROWS, ZROWS)],
                    acc.at[pl.ds(s * ZROWS, ZROWS)])
    plsc.subcore_barrier()
    _agg_pipeline(CH_CS, tbl_hbm, src_hbm.at[c], dst_hbm, s * rpt, s * rpt,
                  rpt, srcv, dstv, buf0, buf1, acc, semg0, semg1, sems0, sems1)
    plsc.subcore_barrier()
    pltpu.sync_copy(acc.at[pl.ds(s * ZROWS, ZROWS)],
                    out_hbm.at[c].at[pl.ds(s * ZROWS, ZROWS)])


@functools.cache
def _agg_cs_call():
    return functools.partial(
        pl.kernel,
        mesh=_mesh(),
        out_type=jax.ShapeDtypeStruct((2, ACC_ROWS, 128), jnp.float32),
        scratch_types=_scratch(CH_CS),
    )(_agg_cs_body)


def _prep_body(deg_ref, x_ref, dinv_ref, z_ref):
    deg = deg_ref[0, :, 0:1] + deg_ref[1, :, 0:1] + 1.0
    dinv = lax.rsqrt(deg)
    dinv_ref[...] = jnp.broadcast_to(dinv, (ROW_BLK, 128))
    z_ref[...] = dinv * x_ref[...]


def _prep_call(degp, x):
    return pl.pallas_call(
        _prep_body,
        grid=(GRID,),
        in_specs=[
            pl.BlockSpec((2, ROW_BLK, 128), lambda i: (0, i, 0)),
            pl.BlockSpec((ROW_BLK, 128), lambda i: (i, 0)),
        ],
        out_specs=[
            pl.BlockSpec((ROW_BLK, 128), lambda i: (i, 0)),
            pl.BlockSpec((ROW_BLK, 128), lambda i: (i, 0)),
        ],
        out_shape=[
            jax.ShapeDtypeStruct((NN, 128), jnp.float32),
            jax.ShapeDtypeStruct((NN, 128), jnp.float32),
        ],
    )(degp, x)


def _layer1_body(acc_ref, z_ref, dinv_ref, w_ref, b_ref, out_ref):
    dinv = dinv_ref[:, 0:1]
    u = dinv * (acc_ref[0] + acc_ref[1] + z_ref[...])
    pre = jnp.dot(u, w_ref[...], preferred_element_type=jnp.float32) + b_ref[0]
    gn = dinv * jnp.maximum(pre, 0.0)
    out_ref[0] = gn[:, :128]
    out_ref[1] = gn[:, 128:]


def _layer1_call(acc, z, dinv128, w, b8):
    return pl.pallas_call(
        _layer1_body,
        grid=(GRID,),
        in_specs=[
            pl.BlockSpec((2, ROW_BLK, 128), lambda i: (0, i, 0)),
            pl.BlockSpec((ROW_BLK, 128), lambda i: (i, 0)),
            pl.BlockSpec((ROW_BLK, 128), lambda i: (i, 0)),
            pl.BlockSpec((128, HID), lambda i: (0, 0)),
            pl.BlockSpec((8, HID), lambda i: (0, 0)),
        ],
        out_specs=pl.BlockSpec((2, ROW_BLK, 128), lambda i: (0, i, 0)),
        out_shape=jax.ShapeDtypeStruct((2, NN, 128), jnp.float32),
    )(acc, z, dinv128, w, b8)


def _layer2_body(acc_ref, g_ref, dinv_ref, w_ref, b_ref, out_ref):
    dinv = dinv_ref[:, 0:1]
    u0 = dinv * (acc_ref[0] + g_ref[0])
    u1 = dinv * (acc_ref[1] + g_ref[1])
    pre = (jnp.dot(u0, w_ref[0], preferred_element_type=jnp.float32)
           + jnp.dot(u1, w_ref[1], preferred_element_type=jnp.float32)
           + b_ref[0])
    gn = dinv * jnp.maximum(pre, 0.0)
    out_ref[0] = gn[:, :128]
    out_ref[1] = gn[:, 128:]


def _layer2_call(acc, g, dinv128, w2, b8):
    return pl.pallas_call(
        _layer2_body,
        grid=(GRID,),
        in_specs=[
            pl.BlockSpec((2, ROW_BLK, 128), lambda i: (0, i, 0)),
            pl.BlockSpec((2, ROW_BLK, 128), lambda i: (0, i, 0)),
            pl.BlockSpec((ROW_BLK, 128), lambda i: (i, 0)),
            pl.BlockSpec((2, 128, HID), lambda i: (0, 0, 0)),
            pl.BlockSpec((8, HID), lambda i: (0, 0)),
        ],
        out_specs=pl.BlockSpec((2, ROW_BLK, 128), lambda i: (0, i, 0)),
        out_shape=jax.ShapeDtypeStruct((2, NN, 128), jnp.float32),
    )(acc, g, dinv128, w2, b8)


def _final_body(acc_ref, g_ref, dinv_ref, w_ref, b_ref, batch_ref,
                wf1_ref, bf1_ref, wf2_ref, bf2_ref,
                sums_ref, counts_ref, out_ref):
    i = pl.program_id(0)
    dinv = dinv_ref[:, 0:1]
    u0 = dinv * (acc_ref[0] + g_ref[0])
    u1 = dinv * (acc_ref[1] + g_ref[1])
    pre = (jnp.dot(u0, w_ref[0], preferred_element_type=jnp.float32)
           + jnp.dot(u1, w_ref[1], preferred_element_type=jnp.float32)
           + b_ref[0])
    h = jnp.maximum(pre, 0.0)
    seg = batch_ref[:, 0:1]
    gid = lax.broadcasted_iota(jnp.int32, (ROW_BLK, NG), 1)
    mask = (seg == gid).astype(jnp.float32)
    psum = lax.dot_general(mask, h, (((0,), (0,)), ((), ())),
                           preferred_element_type=jnp.float32)
    pcnt = lax.dot_general(mask, jnp.ones((ROW_BLK, HID), jnp.float32),
                           (((0,), (0,)), ((), ())),
                           preferred_element_type=jnp.float32)

    @pl.when(i == 0)
    def _():
        sums_ref[...] = jnp.zeros_like(sums_ref)
        counts_ref[...] = jnp.zeros_like(counts_ref)
        out_ref[...] = jnp.zeros_like(out_ref)

    sums_ref[...] += psum
    counts_ref[...] += pcnt

    @pl.when(i == GRID - 1)
    def _():
        pooled = sums_ref[...] / jnp.maximum(counts_ref[...], 1.0)
        t = jnp.maximum(
            jnp.dot(pooled, wf1_ref[...], preferred_element_type=jnp.float32)
            + bf1_ref[0], 0.0)
        out_ref[...] = (jnp.dot(t, wf2_ref[...],
                                preferred_element_type=jnp.float32)
                        + bf2_ref[0, 0])


def _final_call(acc, g, dinv128, w2, b8, batchr, wf1p, bf1p, wf2p, bf2p):
    return pl.pallas_call(
        _final_body,
        grid=(GRID,),
        in_specs=[
            pl.BlockSpec((2, ROW_BLK, 128), lambda i: (0, i, 0)),
            pl.BlockSpec((2, ROW_BLK, 128), lambda i: (0, i, 0)),
            pl.BlockSpec((ROW_BLK, 128), lambda i: (i, 0)),
            pl.BlockSpec((2, 128, HID), lambda i: (0, 0, 0)),
            pl.BlockSpec((8, HID), lambda i: (0, 0)),
            pl.BlockSpec((ROW_BLK, 128), lambda i: (i, 0)),
            pl.BlockSpec((HID, 128), lambda i: (0, 0)),
            pl.BlockSpec((8, 128), lambda i: (0, 0)),
            pl.BlockSpec((128, 128), lambda i: (0, 0)),
            pl.BlockSpec((8, 128), lambda i: (0, 0)),
        ],
        out_specs=[
            pl.BlockSpec((NG, HID), lambda i: (0, 0)),
            pl.BlockSpec((NG, HID), lambda i: (0, 0)),
            pl.BlockSpec((NG, 128), lambda i: (0, 0)),
        ],
        out_shape=[
            jax.ShapeDtypeStruct((NG, HID), jnp.float32),
            jax.ShapeDtypeStruct((NG, HID), jnp.float32),
            jax.ShapeDtypeStruct((NG, 128), jnp.float32),
        ],
    )(acc, g, dinv128, w2, b8, batchr, wf1p, bf1p, wf2p, bf2p)


def kernel(x, edge_index, batch, W1, b1, W2, b2, W3, b3, Wf1, bf1, Wf2, bf2):
    f32 = jnp.float32
    src = edge_index[0].astype(jnp.int32)
    dst = edge_index[1].astype(jnp.int32)
    pad = E_PAD - NE
    src_p = jnp.concatenate(
        [src, jnp.zeros((pad,), jnp.int32)]).reshape(ROWS_ALL, EBLK)
    dst_p = jnp.concatenate(
        [dst, jnp.full((pad,), NN, jnp.int32)]).reshape(ROWS_ALL, EBLK)
    src2 = jnp.stack([src_p, src_p + NN])

    zeros128 = jnp.zeros((ACC_ROWS, 128), f32)
    ones128 = jnp.ones((EBLK, 128), f32)

    def b8(b):
        return jnp.broadcast_to(b[None, :], (8, b.shape[0]))

    batchr = jnp.broadcast_to(batch.astype(jnp.int32)[:, None], (NN, 128))
    wf1p = jnp.pad(Wf1, ((0, 0), (0, 128 - Wf1.shape[1])))
    bf1p = b8(jnp.pad(bf1, (0, 128 - bf1.shape[0])))
    wf2p = jnp.pad(Wf2, ((0, 128 - Wf2.shape[0]), (0, 128 - Wf2.shape[1])))
    bf2p = b8(jnp.broadcast_to(bf2, (128,)))

    degp = _deg_call()(dst_p, ones128, zeros128)
    dinv128, z = _prep_call(degp, x)

    acc1 = _agg_es_call()(z, src_p, dst_p, zeros128)
    g2 = _layer1_call(acc1, z, dinv128, W1, b8(b1))

    acc2 = _agg_cs_call()(g2.reshape(2 * NN, 128), src2, dst_p, zeros128)
    g3 = _layer2_call(acc2, g2, dinv128, W2.reshape(2, 128, HID), b8(b2))

    acc3 = _agg_cs_call()(g3.reshape(2 * NN, 128), src2, dst_p, zeros128)
    _, _, out = _final_call(acc3, g3, dinv128, W3.reshape(2, 128, HID),
                            b8(b3), batchr, wf1p, bf1p, wf2p, bf2p)
    return out[:, 0]


# speedup vs baseline: 2.4524x; 2.3524x over previous
import functools

import jax
import jax.numpy as jnp
from jax import lax
from jax.experimental import pallas as pl
from jax.experimental.pallas import tpu as pltpu
from jax.experimental.pallas import tpu_sc as plsc

NN = 10000
NG = 64
HID = 256
NE = 320000
ACC_ROWS = 10240
E_PAD = 327680
EBLK = 128
ROWS_ALL = E_PAD // EBLK
TILES = 16
ZROWS = ACC_ROWS // TILES
ROW_BLK = 2000
GRID = NN // ROW_BLK


def _mesh():
    return plsc.VectorSubcoreMesh(core_axis_name="c", subcore_axis_name="s")


def _deg_body(dst_hbm, ones_hbm, zeros_hbm, out_hbm, dstv, onesv, acc):
    c = lax.axis_index("c")
    s = lax.axis_index("s")
    rpt = ROWS_ALL // 32
    pltpu.sync_copy(zeros_hbm.at[pl.ds(s * ZROWS, ZROWS)],
                    acc.at[pl.ds(s * ZROWS, ZROWS)])
    pltpu.sync_copy(ones_hbm, onesv)
    base = (c * TILES + s) * rpt
    pltpu.sync_copy(dst_hbm.at[pl.ds(base, rpt)], dstv)
    plsc.subcore_barrier()

    def blk(j, carry):
        pltpu.sync_copy(onesv, acc.at[dstv.at[j]], add=True)
        return carry

    lax.fori_loop(0, rpt, blk, 0)
    plsc.subcore_barrier()
    pltpu.sync_copy(acc.at[pl.ds(s * ZROWS, ZROWS)],
                    out_hbm.at[c].at[pl.ds(s * ZROWS, ZROWS)])


@functools.cache
def _deg_call():
    return functools.partial(
        pl.kernel,
        mesh=_mesh(),
        out_type=jax.ShapeDtypeStruct((2, ACC_ROWS, 128), jnp.float32),
        scratch_types=[
            pltpu.VMEM((ROWS_ALL // 32, EBLK), jnp.int32),
            pltpu.VMEM((EBLK, 128), jnp.float32),
            pltpu.VMEM_SHARED((ACC_ROWS, 128), jnp.float32),
        ],
    )(_deg_body)


def _agg_pipeline(ch, tbl_hbm, src_view, dst_view, src_base, dst_base, nrows,
                  srcv, dstv, buf0, buf1, acc, semg0, semg1, sems0, sems1):
    def wg(buf, sem):
        pltpu.make_async_copy(tbl_hbm.at[srcv.at[0]], buf, sem).wait()

    def ws(buf, sem):
        pltpu.make_async_copy(buf, acc.at[dstv.at[0]], sem).wait()

    def chunk(ci, carry):
        pltpu.sync_copy(src_view.at[pl.ds(src_base + ci * ch, ch)], srcv)
        pltpu.sync_copy(dst_view.at[pl.ds(dst_base + ci * ch, ch)], dstv)
        pltpu.async_copy(tbl_hbm.at[srcv.at[0]], buf0, semg0)
        wg(buf0, semg0)
        pltpu.async_copy(buf0, acc.at[dstv.at[0]], sems0, add=True)
        pltpu.async_copy(tbl_hbm.at[srcv.at[1]], buf1, semg1)

        def inner(k, c2):
            j = 2 * k + 1
            wg(buf1, semg1)
            ws(buf0, sems0)
            pltpu.async_copy(buf1, acc.at[dstv.at[j]], sems1, add=True)
            pltpu.async_copy(tbl_hbm.at[srcv.at[j + 1]], buf0, semg0)
            wg(buf0, semg0)
            ws(buf1, sems1)
            pltpu.async_copy(buf0, acc.at[dstv.at[j + 1]], sems0, add=True)
            pltpu.async_copy(tbl_hbm.at[srcv.at[j + 2]], buf1, semg1)
            return c2

        lax.fori_loop(0, ch // 2 - 1, inner, 0)
        wg(buf1, semg1)
        ws(buf0, sems0)
        pltpu.async_copy(buf1, acc.at[dstv.at[ch - 1]], sems1, add=True)
        ws(buf1, sems1)
        return carry

    lax.fori_loop(0, nrows // ch, chunk, 0)


CH_ES = 40
CH_CS = 40


def _scratch(ch):
    return [
        pltpu.VMEM((ch, EBLK), jnp.int32),
        pltpu.VMEM((ch, EBLK), jnp.int32),
        pltpu.VMEM((EBLK, 128), jnp.float32),
        pltpu.VMEM((EBLK, 128), jnp.float32),
        pltpu.VMEM_SHARED((ACC_ROWS, 128), jnp.float32),
        pltpu.SemaphoreType.DMA,
        pltpu.SemaphoreType.DMA,
        pltpu.SemaphoreType.DMA,
        pltpu.SemaphoreType.DMA,
    ]


def _agg_es_body(tbl_hbm, src_hbm, dst_hbm, zeros_hbm, out_hbm,
                 srcv, dstv, buf0, buf1, acc, semg0, semg1, sems0, sems1):
    c = lax.axis_index("c")
    s = lax.axis_index("s")
    rpt = ROWS_ALL // 32
    pltpu.sync_copy(zeros_hbm.at[pl.ds(s * ZROWS, ZROWS)],
                    acc.at[pl.ds(s * ZROWS, ZROWS)])
    plsc.subcore_barrier()
    base = (c * TILES + s) * rpt
    _agg_pipeline(CH_ES, tbl_hbm, src_hbm, dst_hbm, base, base, rpt,
                  srcv, dstv, buf0, buf1, acc, semg0, semg1, sems0, sems1)
    plsc.subcore_barrier()
    pltpu.sync_copy(acc.at[pl.ds(s * ZROWS, ZROWS)],
                    out_hbm.at[c].at[pl.ds(s * ZROWS, ZROWS)])


@functools.cache
def _agg_es_call():
    return functools.partial(
        pl.kernel,
        mesh=_mesh(),
        out_type=jax.ShapeDtypeStruct((2, ACC_ROWS, 128), jnp.float32),
        scratch_types=_scratch(CH_ES),
    )(_agg_es_body)


def _agg_cs_body(tbl_hbm, src_hbm, dst_hbm, zeros_hbm, out_hbm,
                 srcv, dstv, buf0, buf1, acc, semg0, semg1, sems0, sems1):
    c = lax.axis_index("c")
    s = lax.axis_index("s")
    rpt = ROWS_ALL // TILES
    pltpu.sync_copy(zeros_hbm.at[pl.ds(s * ZROWS, ZROWS)],
                    acc.at[pl.ds(s * ZROWS, ZROWS)])
    plsc.subcore_barrier()
    _agg_pipeline(CH_CS, tbl_hbm, src_hbm.at[c], dst_hbm, s * rpt, s * rpt,
                  rpt, srcv, dstv, buf0, buf1, acc, semg0, semg1, sems0, sems1)
    plsc.subcore_barrier()
    pltpu.sync_copy(acc.at[pl.ds(s * ZROWS, ZROWS)],
                    out_hbm.at[c].at[pl.ds(s * ZROWS, ZROWS)])


@functools.cache
def _agg_cs_call():
    return functools.partial(
        pl.kernel,
        mesh=_mesh(),
        out_type=jax.ShapeDtypeStruct((2, ACC_ROWS, 128), jnp.float32),
        scratch_types=_scratch(CH_CS),
    )(_agg_cs_body)


def _prep_body(deg_ref, x_ref, dinv_ref, z_ref):
    deg = deg_ref[0, :, 0:1] + deg_ref[1, :, 0:1] + 1.0
    dinv = lax.rsqrt(deg)
    dinv_ref[...] = jnp.broadcast_to(dinv, (ROW_BLK, 128))
    z_ref[...] = dinv * x_ref[...]


def _prep_call(degp, x):
    return pl.pallas_call(
        _prep_body,
        grid=(GRID,),
        in_specs=[
            pl.BlockSpec((2, ROW_BLK, 128), lambda i: (0, i, 0)),
            pl.BlockSpec((ROW_BLK, 128), lambda i: (i, 0)),
        ],
        out_specs=[
            pl.BlockSpec((ROW_BLK, 128), lambda i: (i, 0)),
            pl.BlockSpec((ROW_BLK, 128), lambda i: (i, 0)),
        ],
        out_shape=[
            jax.ShapeDtypeStruct((NN, 128), jnp.float32),
            jax.ShapeDtypeStruct((NN, 128), jnp.float32),
        ],
    )(degp, x)


def _layer1_body(acc_ref, z_ref, dinv_ref, w_ref, b_ref, out_ref):
    dinv = dinv_ref[:, 0:1]
    u = dinv * (acc_ref[0] + acc_ref[1] + z_ref[...])
    pre = jnp.dot(u, w_ref[...], preferred_element_type=jnp.float32) + b_ref[0]
    gn = dinv * jnp.maximum(pre, 0.0)
    out_ref[0] = gn[:, :128]
    out_ref[1] = gn[:, 128:]


def _layer1_call(acc, z, dinv128, w, b8):
    return pl.pallas_call(
        _layer1_body,
        grid=(GRID,),
        in_specs=[
            pl.BlockSpec((2, ROW_BLK, 128), lambda i: (0, i, 0)),
            pl.BlockSpec((ROW_BLK, 128), lambda i: (i, 0)),
            pl.BlockSpec((ROW_BLK, 128), lambda i: (i, 0)),
            pl.BlockSpec((128, HID), lambda i: (0, 0)),
            pl.BlockSpec((8, HID), lambda i: (0, 0)),
        ],
        out_specs=pl.BlockSpec((2, ROW_BLK, 128), lambda i: (0, i, 0)),
        out_shape=jax.ShapeDtypeStruct((2, NN, 128), jnp.float32),
    )(acc, z, dinv128, w, b8)


def _layer2_body(acc_ref, g_ref, dinv_ref, w_ref, b_ref, out_ref):
    dinv = dinv_ref[:, 0:1]
    u0 = dinv * (acc_ref[0] + g_ref[0])
    u1 = dinv * (acc_ref[1] + g_ref[1])
    pre = (jnp.dot(u0, w_ref[0], preferred_element_type=jnp.float32)
           + jnp.dot(u1, w_ref[1], preferred_element_type=jnp.float32)
           + b_ref[0])
    gn = dinv * jnp.maximum(pre, 0.0)
    out_ref[0] = gn[:, :128]
    out_ref[1] = gn[:, 128:]


def _layer2_call(acc, g, dinv128, w2, b8):
    return pl.pallas_call(
        _layer2_body,
        grid=(GRID,),
        in_specs=[
            pl.BlockSpec((2, ROW_BLK, 128), lambda i: (0, i, 0)),
            pl.BlockSpec((2, ROW_BLK, 128), lambda i: (0, i, 0)),
            pl.BlockSpec((ROW_BLK, 128), lambda i: (i, 0)),
            pl.BlockSpec((2, 128, HID), lambda i: (0, 0, 0)),
            pl.BlockSpec((8, HID), lambda i: (0, 0)),
        ],
        out_specs=pl.BlockSpec((2, ROW_BLK, 128), lambda i: (0, i, 0)),
        out_shape=jax.ShapeDtypeStruct((2, NN, 128), jnp.float32),
    )(acc, g, dinv128, w2, b8)


def _final_body(acc_ref, g_ref, dinv_ref, w_ref, b_ref, batch_ref,
                wf1_ref, bf1_ref, wf2_ref, bf2_ref,
                sums_ref, counts_ref, out_ref):
    i = pl.program_id(0)
    dinv = dinv_ref[:, 0:1]
    u0 = dinv * (acc_ref[0] + g_ref[0])
    u1 = dinv * (acc_ref[1] + g_ref[1])
    pre = (jnp.dot(u0, w_ref[0], preferred_element_type=jnp.float32)
           + jnp.dot(u1, w_ref[1], preferred_element_type=jnp.float32)
           + b_ref[0])
    h = jnp.maximum(pre, 0.0)
    seg = batch_ref[:, 0:1]
    gid = lax.broadcasted_iota(jnp.int32, (ROW_BLK, NG), 1)
    mask = (seg == gid).astype(jnp.float32)
    psum = lax.dot_general(mask, h, (((0,), (0,)), ((), ())),
                           preferred_element_type=jnp.float32)
    pcnt = lax.dot_general(mask, jnp.ones((ROW_BLK, HID), jnp.float32),
                           (((0,), (0,)), ((), ())),
                           preferred_element_type=jnp.float32)

    @pl.when(i == 0)
    def _():
        sums_ref[...] = jnp.zeros_like(sums_ref)
        counts_ref[...] = jnp.zeros_like(counts_ref)
        out_ref[...] = jnp.zeros_like(out_ref)

    sums_ref[...] += psum
    counts_ref[...] += pcnt

    @pl.when(i == GRID - 1)
    def _():
        pooled = sums_ref[...] / jnp.maximum(counts_ref[...], 1.0)
        t = jnp.maximum(
            jnp.dot(pooled, wf1_ref[...], preferred_element_type=jnp.float32)
            + bf1_ref[0], 0.0)
        out_ref[...] = (jnp.dot(t, wf2_ref[...],
                                preferred_element_type=jnp.float32)
                        + bf2_ref[0, 0])


def _final_call(acc, g, dinv128, w2, b8, batchr, wf1p, bf1p, wf2p, bf2p):
    return pl.pallas_call(
        _final_body,
        grid=(GRID,),
        in_specs=[
            pl.BlockSpec((2, ROW_BLK, 128), lambda i: (0, i, 0)),
            pl.BlockSpec((2, ROW_BLK, 128), lambda i: (0, i, 0)),
            pl.BlockSpec((ROW_BLK, 128), lambda i: (i, 0)),
            pl.BlockSpec((2, 128, HID), lambda i: (0, 0, 0)),
            pl.BlockSpec((8, HID), lambda i: (0, 0)),
            pl.BlockSpec((ROW_BLK, 128), lambda i: (i, 0)),
            pl.BlockSpec((HID, 128), lambda i: (0, 0)),
            pl.BlockSpec((8, 128), lambda i: (0, 0)),
            pl.BlockSpec((128, 128), lambda i: (0, 0)),
            pl.BlockSpec((8, 128), lambda i: (0, 0)),
        ],
        out_specs=[
            pl.BlockSpec((NG, HID), lambda i: (0, 0)),
            pl.BlockSpec((NG, HID), lambda i: (0, 0)),
            pl.BlockSpec((NG, 128), lambda i: (0, 0)),
        ],
        out_shape=[
            jax.ShapeDtypeStruct((NG, HID), jnp.float32),
            jax.ShapeDtypeStruct((NG, HID), jnp.float32),
            jax.ShapeDtypeStruct((NG, 128), jnp.float32),
        ],
    )(acc, g, dinv128, w2, b8, batchr, wf1p, bf1p, wf2p, bf2p)


def kernel(x, edge_index, batch, W1, b1, W2, b2, W3, b3, Wf1, bf1, Wf2, bf2):
    f32 = jnp.float32
    src = edge_index[0].astype(jnp.int32)
    dst = edge_index[1].astype(jnp.int32)
    pad = E_PAD - NE
    iota = jnp.arange(pad, dtype=jnp.int32)
    src_p = jnp.concatenate(
        [src, iota % NN]).reshape(ROWS_ALL, EBLK)
    dst_p = jnp.concatenate(
        [dst, NN + (iota % (ACC_ROWS - NN))]).reshape(ROWS_ALL, EBLK)
    src2 = jnp.stack([src_p, src_p + NN])

    zeros128 = jnp.zeros((ACC_ROWS, 128), f32)
    ones128 = jnp.ones((EBLK, 128), f32)

    def b8(b):
        return jnp.broadcast_to(b[None, :], (8, b.shape[0]))

    batchr = jnp.broadcast_to(batch.astype(jnp.int32)[:, None], (NN, 128))
    wf1p = jnp.pad(Wf1, ((0, 0), (0, 128 - Wf1.shape[1])))
    bf1p = b8(jnp.pad(bf1, (0, 128 - bf1.shape[0])))
    wf2p = jnp.pad(Wf2, ((0, 128 - Wf2.shape[0]), (0, 128 - Wf2.shape[1])))
    bf2p = b8(jnp.broadcast_to(bf2, (128,)))

    degp = _deg_call()(dst_p, ones128, zeros128)
    dinv128, z = _prep_call(degp, x)

    acc1 = _agg_es_call()(z, src_p, dst_p, zeros128)
    g2 = _layer1_call(acc1, z, dinv128, W1, b8(b1))

    acc2 = _agg_cs_call()(g2.reshape(2 * NN, 128), src2, dst_p, zeros128)
    g3 = _layer2_call(acc2, g2, dinv128, W2.reshape(2, 128, HID), b8(b2))

    acc3 = _agg_cs_call()(g3.reshape(2 * NN, 128), src2, dst_p, zeros128)
    _, _, out = _final_call(acc3, g3, dinv128, W3.reshape(2, 128, HID),
                            b8(b3), batchr, wf1p, bf1p, wf2p, bf2p)
    return out[:, 0]
